# Initial kernel scaffold; baseline (speedup 1.0000x reference)
import jax
import jax.numpy as jnp
from jax.experimental import pallas as pl


def kernel(x, batch, W1, b1, W2, b2):
    raise NotImplementedError("write your pallas kernel here")


# speedup vs baseline: 10.6171x; 10.6171x over previous
import jax
import jax.numpy as jnp
from jax.experimental import pallas as pl
from jax.experimental.pallas import tpu as pltpu

NG = 128
ROWS = 2000


def _body(x_ref, ids_ref, W1_ref, b1_ref, W2_ref, b2_ref, out_ref,
          m_ref, d_ref, acc_ref):
    k = pl.program_id(0)
    nb = pl.num_programs(0)
    neg = jnp.float32(-jnp.inf)

    @pl.when(k == 0)
    def _init():
        m_ref[...] = jnp.full((1, NG), neg, jnp.float32)
        d_ref[...] = jnp.zeros((1, NG), jnp.float32)
        acc_ref[...] = jnp.zeros(acc_ref.shape, jnp.float32)

    x = x_ref[...]
    ids = ids_ref[...]
    h = jnp.tanh(jnp.dot(x, W1_ref[...], preferred_element_type=jnp.float32)
                 + b1_ref[...])
    s = (jnp.sum(h * W2_ref[...], axis=1, keepdims=True)
         + b2_ref[...])

    lanes = jax.lax.broadcasted_iota(jnp.int32, (ROWS, NG), 1)
    pb = lanes == ids
    pf = pb.astype(jnp.float32)

    mx = jnp.max(jnp.where(pb, s, neg), axis=0, keepdims=True)
    m_old = m_ref[...]
    m_new = jnp.maximum(m_old, mx)
    scale = jnp.where(m_old == neg, 0.0, jnp.exp(m_old - m_new))
    m_ref[...] = m_new

    m_sel = jnp.sum(jnp.where(pb, m_new, 0.0), axis=1, keepdims=True)
    e = jnp.exp(s - m_sel)
    d_ref[...] = d_ref[...] * scale + jnp.sum(jnp.where(pb, e, 0.0),
                                              axis=0, keepdims=True)

    wx = x * e
    upd = jax.lax.dot_general(wx, pf, (((0,), (0,)), ((), ())),
                              preferred_element_type=jnp.float32)
    acc_ref[...] = acc_ref[...] * scale + upd

    @pl.when(k == nb - 1)
    def _fin():
        d = d_ref[...]
        safe = jnp.where(d == 0.0, 1.0, d)
        accn = jnp.where(d == 0.0, 0.0, acc_ref[...] / safe)
        eye = (jax.lax.broadcasted_iota(jnp.int32, (NG, NG), 0)
               == jax.lax.broadcasted_iota(jnp.int32, (NG, NG), 1)
               ).astype(jnp.float32)
        out_ref[...] = jax.lax.dot_general(accn, eye, (((0,), (0,)), ((), ())),
                                           preferred_element_type=jnp.float32)


def kernel(x, batch, W1, b1, W2, b2):
    N, d = x.shape
    nb = N // ROWS
    ids2 = batch.reshape(N, 1)
    b1r = b1.reshape(1, -1)
    w2r = W2.reshape(1, -1)
    b2r = b2.reshape(1, 1)
    return pl.pallas_call(
        _body,
        grid=(nb,),
        in_specs=[
            pl.BlockSpec((ROWS, d), lambda k: (k, 0)),
            pl.BlockSpec((ROWS, 1), lambda k: (k, 0)),
            pl.BlockSpec((d, d // 2), lambda k: (0, 0)),
            pl.BlockSpec((1, d // 2), lambda k: (0, 0)),
            pl.BlockSpec((1, d // 2), lambda k: (0, 0)),
            pl.BlockSpec((1, 1), lambda k: (0, 0)),
        ],
        out_specs=pl.BlockSpec((NG, NG), lambda k: (0, 0)),
        out_shape=jax.ShapeDtypeStruct((NG, NG), jnp.float32),
        scratch_shapes=[
            pltpu.VMEM((1, NG), jnp.float32),
            pltpu.VMEM((1, NG), jnp.float32),
            pltpu.VMEM((d, NG), jnp.float32),
        ],
        compiler_params=pltpu.CompilerParams(
            dimension_semantics=("arbitrary",)),
    )(x, ids2, W1, b1r, w2r, b2r)


# speedup vs baseline: 14.5286x; 1.3684x over previous
import jax
import jax.numpy as jnp
from jax.experimental import pallas as pl
from jax.experimental.pallas import tpu as pltpu

NG = 128
ROWS = 2000


def _body(x_ref, ids_ref, W1_ref, b1_ref, w2_ref, b2e_ref, out_ref,
          d_ref, acc_ref):
    k = pl.program_id(0)
    nb = pl.num_programs(0)

    @pl.when(k == 0)
    def _init():
        d_ref[...] = jnp.zeros(d_ref.shape, jnp.float32)
        acc_ref[...] = jnp.zeros(acc_ref.shape, jnp.float32)

    x = x_ref[...]
    ids = ids_ref[...]
    h = jnp.tanh(jnp.dot(x, W1_ref[...], preferred_element_type=jnp.float32)
                 + b1_ref[...])
    s = (jnp.dot(h, w2_ref[...], preferred_element_type=jnp.float32)
         + b2e_ref[...])
    e = jnp.exp(s)

    lanes = jax.lax.broadcasted_iota(jnp.int32, (ROWS, NG), 1)
    em = jnp.where(lanes == ids, e, 0.0)
    d_ref[...] += jnp.sum(em, axis=0, keepdims=True)
    acc_ref[...] += jax.lax.dot_general(x, em, (((0,), (0,)), ((), ())),
                                        preferred_element_type=jnp.float32)

    @pl.when(k == nb - 1)
    def _fin():
        eye = (jax.lax.broadcasted_iota(jnp.int32, (NG, NG), 0)
               == jax.lax.broadcasted_iota(jnp.int32, (NG, NG), 1)
               ).astype(jnp.float32)
        accT = jax.lax.dot_general(acc_ref[...], eye, (((0,), (0,)), ((), ())),
                                   preferred_element_type=jnp.float32)
        dcol = jax.lax.dot_general(eye, d_ref[...], (((1,), (1,)), ((), ())),
                                   preferred_element_type=jnp.float32)
        out_ref[...] = jnp.where(dcol == 0.0, 0.0, accT / dcol)


def kernel(x, batch, W1, b1, W2, b2):
    N, d = x.shape
    nb = N // ROWS
    ids2 = batch.reshape(N, 1)
    b1r = b1.reshape(1, -1)
    w2r = W2
    bound = jnp.sum(jnp.abs(W2)) + jnp.abs(b2[0])
    b2er = (b2 - bound).reshape(1, 1)
    return pl.pallas_call(
        _body,
        grid=(nb,),
        in_specs=[
            pl.BlockSpec((ROWS, d), lambda k: (k, 0)),
            pl.BlockSpec((ROWS, 1), lambda k: (k, 0)),
            pl.BlockSpec((d, d // 2), lambda k: (0, 0)),
            pl.BlockSpec((1, d // 2), lambda k: (0, 0)),
            pl.BlockSpec((d // 2, 1), lambda k: (0, 0)),
            pl.BlockSpec((1, 1), lambda k: (0, 0)),
        ],
        out_specs=pl.BlockSpec((NG, NG), lambda k: (0, 0)),
        out_shape=jax.ShapeDtypeStruct((NG, NG), jnp.float32),
        scratch_shapes=[
            pltpu.VMEM((1, NG), jnp.float32),
            pltpu.VMEM((d, NG), jnp.float32),
        ],
        compiler_params=pltpu.CompilerParams(
            dimension_semantics=("arbitrary",)),
    )(x, ids2, W1, b1r, w2r, b2er)


# speedup vs baseline: 25.8948x; 1.7823x over previous
import jax
import jax.numpy as jnp
from jax.experimental import pallas as pl
from jax.experimental.pallas import tpu as pltpu

NG = 128
ROWS = 2000


def _body(x_ref, ids_ref, W1_ref, b1_ref, W2_ref, b2e_ref, out_ref,
          d_ref, acc_ref):
    k = pl.program_id(0)
    nb = pl.num_programs(0)

    @pl.when(k == 0)
    def _init():
        d_ref[...] = jnp.zeros(d_ref.shape, jnp.float32)
        acc_ref[...] = jnp.zeros(acc_ref.shape, jnp.float32)

    x = x_ref[...]
    ids = ids_ref[0]
    h = jnp.tanh(jnp.dot(x, W1_ref[...], preferred_element_type=jnp.float32)
                 + b1_ref[...])
    s = (jnp.dot(h, W2_ref[...], preferred_element_type=jnp.float32)
         + b2e_ref[...])
    e = jnp.exp(s)

    g_iota = jax.lax.broadcasted_iota(jnp.int32, (NG, ROWS), 0)
    pbf = jnp.where(ids == g_iota, 1.0, 0.0)
    d_ref[...] += jnp.dot(pbf, e, preferred_element_type=jnp.float32)
    wx = x * e
    acc_ref[...] += jnp.dot(pbf, wx, preferred_element_type=jnp.float32)

    @pl.when(k == nb - 1)
    def _fin():
        dcol = d_ref[...]
        out_ref[...] = jnp.where(dcol == 0.0, 0.0, acc_ref[...] / dcol)


def kernel(x, batch, W1, b1, W2, b2):
    N, d = x.shape
    nb = N // ROWS
    ids3 = batch.reshape(nb, 1, ROWS)
    b1r = b1.reshape(1, -1)
    bound = jnp.sum(jnp.abs(W2)) + jnp.abs(b2[0])
    b2er = (b2 - bound).reshape(1, 1)
    return pl.pallas_call(
        _body,
        grid=(nb,),
        in_specs=[
            pl.BlockSpec((ROWS, d), lambda k: (k, 0)),
            pl.BlockSpec((1, 1, ROWS), lambda k: (k, 0, 0)),
            pl.BlockSpec((d, d // 2), lambda k: (0, 0)),
            pl.BlockSpec((1, d // 2), lambda k: (0, 0)),
            pl.BlockSpec((d // 2, 1), lambda k: (0, 0)),
            pl.BlockSpec((1, 1), lambda k: (0, 0)),
        ],
        out_specs=pl.BlockSpec((NG, d), lambda k: (0, 0)),
        out_shape=jax.ShapeDtypeStruct((NG, d), jnp.float32),
        scratch_shapes=[
            pltpu.VMEM((NG, 1), jnp.float32),
            pltpu.VMEM((NG, d), jnp.float32),
        ],
        compiler_params=pltpu.CompilerParams(
            dimension_semantics=("arbitrary",)),
    )(x, ids3, W1, b1r, W2, b2er)


# speedup vs baseline: 34.6703x; 1.3389x over previous
import jax
import jax.numpy as jnp
from jax.experimental import pallas as pl
from jax.experimental.pallas import tpu as pltpu

NG = 128
ROWS = 4000


def _body(x_ref, ids_ref, W1_ref, b1_ref, W2_ref, b2e_ref, out_ref,
          d_ref, acc_ref):
    k = pl.program_id(0)
    nb = pl.num_programs(0)

    @pl.when(k == 0)
    def _init():
        d_ref[...] = jnp.zeros(d_ref.shape, jnp.float32)
        acc_ref[...] = jnp.zeros(acc_ref.shape, jnp.float32)

    x = x_ref[...]
    xb = x.astype(jnp.bfloat16)
    ids = ids_ref[0]
    h = jnp.tanh(jnp.dot(xb, W1_ref[...], preferred_element_type=jnp.float32)
                 + b1_ref[...])
    s = (jnp.dot(h.astype(jnp.bfloat16), W2_ref[...],
                 preferred_element_type=jnp.float32)
         + b2e_ref[...])
    e = jnp.exp(s)
    eb = e.astype(jnp.bfloat16)

    g_iota = jax.lax.broadcasted_iota(jnp.int32, (NG, ROWS), 0)
    pbf = jnp.where(ids == g_iota, 1.0, 0.0).astype(jnp.bfloat16)
    d_ref[...] += jnp.dot(pbf, eb, preferred_element_type=jnp.float32)
    wx = xb * eb
    acc_ref[...] += jnp.dot(pbf, wx, preferred_element_type=jnp.float32)

    @pl.when(k == nb - 1)
    def _fin():
        dcol = d_ref[...]
        out_ref[...] = jnp.where(dcol == 0.0, 0.0, acc_ref[...] / dcol)


def kernel(x, batch, W1, b1, W2, b2):
    N, d = x.shape
    nb = N // ROWS
    ids3 = batch.reshape(nb, 1, ROWS)
    b1r = b1.reshape(1, -1)
    bound = jnp.sum(jnp.abs(W2)) + jnp.abs(b2[0])
    b2er = (b2 - bound).reshape(1, 1)
    return pl.pallas_call(
        _body,
        grid=(nb,),
        in_specs=[
            pl.BlockSpec((ROWS, d), lambda k: (k, 0)),
            pl.BlockSpec((1, 1, ROWS), lambda k: (k, 0, 0)),
            pl.BlockSpec((d, d // 2), lambda k: (0, 0)),
            pl.BlockSpec((1, d // 2), lambda k: (0, 0)),
            pl.BlockSpec((d // 2, 1), lambda k: (0, 0)),
            pl.BlockSpec((1, 1), lambda k: (0, 0)),
        ],
        out_specs=pl.BlockSpec((NG, d), lambda k: (0, 0)),
        out_shape=jax.ShapeDtypeStruct((NG, d), jnp.float32),
        scratch_shapes=[
            pltpu.VMEM((NG, 1), jnp.float32),
            pltpu.VMEM((NG, d), jnp.float32),
        ],
        compiler_params=pltpu.CompilerParams(
            dimension_semantics=("arbitrary",)),
    )(x, ids3, W1.astype(jnp.bfloat16), b1r, W2.astype(jnp.bfloat16), b2er)
